# baseline (speedup 1.0000x reference)
import functools

import jax
import jax.numpy as jnp
from jax.experimental import pallas as pl
from jax.experimental.pallas import tpu as pltpu

_BIG = 1e9


def _min2_tree(u):
    w = u.shape[1]
    h = w // 2
    a, b = u[:, :h], u[:, h:]
    m1 = jnp.minimum(a, b)
    m2 = jnp.maximum(a, b)
    w = h
    while w > 1:
        h = w // 2
        a1, b1 = m1[:, :h], m1[:, h:]
        a2, b2 = m2[:, :h], m2[:, h:]
        m1, m2 = (
            jnp.minimum(a1, b1),
            jnp.minimum(jnp.maximum(a1, b1), jnp.minimum(a2, b2)),
        )
        w = h
    return m1, m2


def _make_aug(x_block):
    sq = jnp.sum(x_block * x_block, axis=1, keepdims=True)
    return jnp.concatenate([x_block * -2.0, jnp.ones_like(sq), sq], axis=1)


def _process(buf_ref, blk, valid, pen_ref, k):
    rpb = pen_ref.shape[0]
    buf_ref[:, pl.ds(blk * rpb, rpb)] += pen_ref[:]
    u = jnp.maximum(buf_ref[:, :], 0.0)
    m1q, m2q = _min2_tree(u)
    s = jnp.exp(-jnp.sqrt(u))
    denom = jnp.sum(s, axis=1, keepdims=True)
    loss_rows = (jnp.sqrt(m1q) + jnp.sqrt(m2q)) * (1.0 / k) + jnp.log(denom)
    part = jnp.sum(loss_rows)[None, None]
    return jnp.where(valid, part, 0.0)


def _knn_loss_step(xa_ref, xb_ref, xt_ref, pen_ref, out_ref,
                   xt_aug_ref, bufa_ref, bufb_ref, *, k, rows_per_blk, nblk):
    s = pl.program_id(0)
    dim = xt_ref.shape[0]

    @pl.when(s == 0)
    def _init():
        xt = xt_ref[:]
        xt_aug_ref[0:dim, :] = xt
        xt_aug_ref[dim:dim + 1, :] = jnp.sum(xt * xt, axis=0, keepdims=True)
        xt_aug_ref[dim + 1:dim + 2, :] = jnp.ones_like(
            xt_aug_ref[dim + 1:dim + 2, :])
        bufb_ref[:, :] = jnp.zeros_like(bufb_ref)
        out_ref[:, :] = jnp.zeros((1, 1), jnp.float32)

    dn = (((1,), (0,)), ((), ()))
    blk_a = jnp.minimum(2 * s, nblk - 1)
    blk_b1 = jnp.maximum(2 * s - 1, 0)
    blk_b2 = jnp.minimum(2 * s + 1, nblk - 1)

    bufa_ref[:, :] = jax.lax.dot_general(
        _make_aug(xa_ref[:]), xt_aug_ref[:], dn,
        preferred_element_type=jnp.float32)
    acc = _process(bufb_ref, blk_b1, s > 0, pen_ref, k)
    bufb_ref[:, :] = jax.lax.dot_general(
        _make_aug(xb_ref[:]), xt_aug_ref[:], dn,
        preferred_element_type=jnp.float32)
    acc += _process(bufa_ref, blk_a, 2 * s < nblk, pen_ref, k)

    out_ref[:, :] += acc


def kernel(x):
    n, d = x.shape
    rows_per_blk = 512
    nblk = n // rows_per_blk
    pen = _BIG * jnp.eye(rows_per_blk, dtype=jnp.float32)
    grid = (nblk // 2 + 1,)
    out = pl.pallas_call(
        functools.partial(_knn_loss_step, k=2, rows_per_blk=rows_per_blk,
                          nblk=nblk),
        grid=grid,
        in_specs=[
            pl.BlockSpec((rows_per_blk, d),
                         lambda s: (jnp.minimum(2 * s, nblk - 1), 0)),
            pl.BlockSpec((rows_per_blk, d),
                         lambda s: (jnp.minimum(2 * s + 1, nblk - 1), 0)),
            pl.BlockSpec((d, n), lambda s: (0, 0)),
            pl.BlockSpec((rows_per_blk, rows_per_blk), lambda s: (0, 0)),
        ],
        out_specs=pl.BlockSpec((1, 1), lambda s: (0, 0)),
        out_shape=jax.ShapeDtypeStruct((1, 1), jnp.float32),
        scratch_shapes=[
            pltpu.VMEM((d + 2, n), jnp.float32),
            pltpu.VMEM((rows_per_blk, n), jnp.float32),
            pltpu.VMEM((rows_per_blk, n), jnp.float32),
        ],
    )(x, x, x.T, pen)
    return out[0, 0] / n
